# baseline (speedup 1.0000x reference)
import jax
import jax.numpy as jnp
from jax.experimental import pallas as pl
from jax.experimental.pallas import tpu as pltpu

_NUM_BINS = 15
_LANES = 128


def _ece_body(nb, num_bins, bnd_ref, sm_ref, lab_ref, out_ref,
              cnt_ref, acc_ref, cfs_ref):
    j = pl.program_id(1)

    @pl.when(j == 0)
    def _init():
        cnt_ref[...] = jnp.zeros_like(cnt_ref)
        acc_ref[...] = jnp.zeros_like(acc_ref)
        cfs_ref[...] = jnp.zeros_like(cfs_ref)

    c_dim = sm_ref.shape[1]
    best = sm_ref[0, 0]
    besti = jnp.zeros(best.shape, jnp.int32)
    for c in range(1, c_dim):
        xc = sm_ref[0, c]
        gt = xc > best
        best = jnp.where(gt, xc, best)
        besti = jnp.where(gt, c, besti)

    corr = (besti == lab_ref[0]).astype(jnp.float32)

    for i in range(num_bins + 1):
        m = (best > bnd_ref[i]).astype(jnp.float32)
        cnt_ref[i, :] += jnp.sum(m, axis=0)
        acc_ref[i, :] += jnp.sum(m * corr, axis=0)
        cfs_ref[i, :] += jnp.sum(m * best, axis=0)

    @pl.when(j == nb - 1)
    def _fin():
        n_total = nb * sm_ref.shape[2] * sm_ref.shape[3]
        cnt = jnp.sum(cnt_ref[...], axis=1, keepdims=True)
        accs = jnp.sum(acc_ref[...], axis=1, keepdims=True)
        cfss = jnp.sum(cfs_ref[...], axis=1, keepdims=True)
        count = cnt[:-1] - cnt[1:]
        prop = count / float(n_total)
        denom = jnp.maximum(count, 1.0)
        acc_b = (accs[:-1] - accs[1:]) / denom
        cfs_b = (cfss[:-1] - cfss[1:]) / denom
        contrib = jnp.where(count > 0.0,
                            jnp.abs(cfs_b - acc_b) * prop, 0.0)
        out_ref[0, 0, :] = jnp.full((_LANES,), jnp.sum(contrib), jnp.float32)


def kernel(edl_u, softmax, label):
    del edl_u
    b_dim, c_dim, n = softmax.shape
    nr = n // _LANES
    r = next(x for x in (128, 64, 32, 16, 8, 4, 2, 1) if nr % x == 0)
    nb = nr // r

    sm4 = softmax.reshape(b_dim, c_dim, nr, _LANES)
    lab3 = label.astype(jnp.int32).reshape(b_dim, nr, _LANES)
    bnd = jnp.linspace(0.0, 1.0, _NUM_BINS + 1, dtype=jnp.float32)

    import functools
    body = functools.partial(_ece_body, nb, _NUM_BINS)
    out = pl.pallas_call(
        body,
        grid=(b_dim, nb),
        in_specs=[
            pl.BlockSpec(memory_space=pltpu.SMEM),
            pl.BlockSpec((1, c_dim, r, _LANES), lambda b, j: (b, 0, j, 0)),
            pl.BlockSpec((1, r, _LANES), lambda b, j: (b, j, 0)),
        ],
        out_specs=pl.BlockSpec((1, 1, _LANES), lambda b, j: (b, 0, 0)),
        out_shape=jax.ShapeDtypeStruct((b_dim, 1, _LANES), jnp.float32),
        scratch_shapes=[pltpu.VMEM((_NUM_BINS + 1, _LANES), jnp.float32)
                        for _ in range(3)],
    )(bnd, sm4, lab3)
    return out[:, 0, 0]


# speedup vs baseline: 1.3548x; 1.3548x over previous
import functools

import jax
import jax.numpy as jnp
from jax.experimental import pallas as pl
from jax.experimental.pallas import tpu as pltpu

_NUM_BINS = 15
_LANES = 128


def _ece_body(nb, c_dim, n_total, bnd_ref, sm_ref, lab_ref, out_ref,
              cnt_ref, acc_ref, cfs_ref):
    j = pl.program_id(0)
    b = pl.program_id(1)

    @pl.when(j == 0)
    def _init():
        zeros = jnp.zeros(cnt_ref.shape[1:], jnp.float32)
        cnt_ref[b] = zeros
        acc_ref[b] = zeros
        cfs_ref[b] = zeros

    x = sm_ref[0]
    t = x.shape[1]
    conf1 = jnp.max(x, axis=0)
    rev = jax.lax.broadcasted_iota(jnp.int32, (c_dim, 1), 0)
    rev = float(c_dim - 1) - rev.astype(jnp.float32)
    s1 = jnp.max(jnp.where(x == conf1[None, :], rev, -1.0), axis=0)
    lab1 = lab_ref[b]

    rows = t // (8 * _LANES)
    conf = conf1.reshape(rows, 8, _LANES)
    s3 = s1.reshape(rows, 8, _LANES)
    lab3 = lab1.reshape(rows, 8, _LANES).astype(jnp.float32)
    corr = (s3 == float(c_dim - 1) - lab3).astype(jnp.float32)

    for i in range(_NUM_BINS + 1):
        m = conf > bnd_ref[i]
        m_f = m.astype(jnp.float32)
        cnt_ref[b, i] += jnp.sum(m_f, axis=0)
        acc_ref[b, i] += jnp.sum(jnp.where(m, corr, 0.0), axis=0)
        cfs_ref[b, i] += jnp.sum(jnp.where(m, conf, 0.0), axis=0)

    @pl.when(j == nb - 1)
    def _fin():
        cnt = jnp.sum(jnp.sum(cnt_ref[b], axis=1), axis=1, keepdims=True)
        accs = jnp.sum(jnp.sum(acc_ref[b], axis=1), axis=1, keepdims=True)
        cfss = jnp.sum(jnp.sum(cfs_ref[b], axis=1), axis=1, keepdims=True)
        count = cnt[:-1] - cnt[1:]
        prop = count / float(n_total)
        denom = jnp.maximum(count, 1.0)
        acc_b = (accs[:-1] - accs[1:]) / denom
        cfs_b = (cfss[:-1] - cfss[1:]) / denom
        contrib = jnp.where(count > 0.0,
                            jnp.abs(cfs_b - acc_b) * prop, 0.0)
        out_ref[0, 0, :] = jnp.full((_LANES,), jnp.sum(contrib), jnp.float32)


def kernel(edl_u, softmax, label):
    del edl_u
    b_dim, c_dim, n = softmax.shape
    tile = 16384
    while n % tile:
        tile //= 2
    nb = n // tile

    label = label.astype(jnp.int32)
    bnd = jnp.linspace(0.0, 1.0, _NUM_BINS + 1, dtype=jnp.float32)

    body = functools.partial(_ece_body, nb, c_dim, n)
    out = pl.pallas_call(
        body,
        grid=(nb, b_dim),
        in_specs=[
            pl.BlockSpec(memory_space=pltpu.SMEM),
            pl.BlockSpec((1, c_dim, tile), lambda j, b: (b, 0, j)),
            pl.BlockSpec((b_dim, tile), lambda j, b: (0, j)),
        ],
        out_specs=pl.BlockSpec((1, 1, _LANES), lambda j, b: (b, 0, 0)),
        out_shape=jax.ShapeDtypeStruct((b_dim, 1, _LANES), jnp.float32),
        scratch_shapes=[pltpu.VMEM((b_dim, _NUM_BINS + 1, 8, _LANES),
                                   jnp.float32) for _ in range(3)],
    )(bnd, softmax, label)
    return out[:, 0, 0]


# speedup vs baseline: 3.7737x; 2.7854x over previous
import functools

import jax
import jax.numpy as jnp
from jax.experimental import pallas as pl
from jax.experimental.pallas import tpu as pltpu

_NUM_BINS = 15
_LANES = 128


def _lane_fold(a):
    t = a.shape[1]
    while t > _LANES:
        half = t // 2
        a = a[:, :half] + a[:, half:]
        t = half
    return a


def _ece_body(nb, c_dim, n_total, bnd_ref, sm_ref, lab_ref, out_ref,
              cnt_ref, acc_ref, cfs_ref):
    j = pl.program_id(0)

    @pl.when(j == 0)
    def _init():
        zeros = jnp.zeros(cnt_ref.shape, jnp.float32)
        cnt_ref[...] = zeros
        acc_ref[...] = zeros
        cfs_ref[...] = zeros

    best = sm_ref[0]
    sbest = jnp.full(best.shape, float(c_dim - 1), jnp.float32)
    for c in range(1, c_dim):
        xc = sm_ref[c]
        gt = xc > best
        best = jnp.maximum(xc, best)
        sbest = jnp.where(gt, float(c_dim - 1 - c), sbest)

    target = float(c_dim - 1) - lab_ref[...].astype(jnp.float32)
    corr = (sbest == target).astype(jnp.float32)

    for i in range(_NUM_BINS + 1):
        m = best > bnd_ref[i]
        cnt_ref[i] += _lane_fold(m.astype(jnp.float32))
        acc_ref[i] += _lane_fold(jnp.where(m, corr, 0.0))
        cfs_ref[i] += _lane_fold(jnp.where(m, best, 0.0))

    @pl.when(j == nb - 1)
    def _fin():
        cnt = jnp.sum(cnt_ref[...], axis=2)
        accs = jnp.sum(acc_ref[...], axis=2)
        cfss = jnp.sum(cfs_ref[...], axis=2)
        count = cnt[:-1] - cnt[1:]
        prop = count / float(n_total)
        denom = jnp.maximum(count, 1.0)
        acc_b = (accs[:-1] - accs[1:]) / denom
        cfs_b = (cfss[:-1] - cfss[1:]) / denom
        contrib = jnp.where(count > 0.0,
                            jnp.abs(cfs_b - acc_b) * prop, 0.0)
        ece = jnp.sum(contrib, axis=0)
        out_ref[...] = jnp.broadcast_to(ece[:, None], out_ref.shape)


def kernel(edl_u, softmax, label):
    del edl_u
    b_dim, c_dim, n = softmax.shape
    sm_t = jnp.transpose(softmax, (1, 0, 2))
    tile = 4096
    while n % tile:
        tile //= 2
    nb = n // tile

    label = label.astype(jnp.int32)
    bnd = jnp.linspace(0.0, 1.0, _NUM_BINS + 1, dtype=jnp.float32)

    body = functools.partial(_ece_body, nb, c_dim, n)
    out = pl.pallas_call(
        body,
        grid=(nb,),
        in_specs=[
            pl.BlockSpec(memory_space=pltpu.SMEM),
            pl.BlockSpec((c_dim, b_dim, tile), lambda j: (0, 0, j)),
            pl.BlockSpec((b_dim, tile), lambda j: (0, j)),
        ],
        out_specs=pl.BlockSpec((b_dim, _LANES), lambda j: (0, 0)),
        out_shape=jax.ShapeDtypeStruct((b_dim, _LANES), jnp.float32),
        scratch_shapes=[pltpu.VMEM((_NUM_BINS + 1, b_dim, _LANES),
                                   jnp.float32) for _ in range(3)],
    )(bnd, sm_t, label)
    return out[:, 0]


# speedup vs baseline: 4.2349x; 1.1222x over previous
import functools

import jax
import jax.numpy as jnp
from jax.experimental import pallas as pl
from jax.experimental.pallas import tpu as pltpu

_NUM_BINS = 15
_LANES = 128


def _lane_fold(a):
    t = a.shape[1]
    while t > _LANES:
        half = t // 2
        a = a[:, :half] + a[:, half:]
        t = half
    return a


def _ece_body(nb, c_dim, n_total, bnd_ref, sm_ref, lab_ref, out_ref,
              cnt_ref, acc_ref, cfs_ref):
    j = pl.program_id(0)

    @pl.when(j == 0)
    def _init():
        zeros = jnp.zeros(cnt_ref.shape, jnp.float32)
        cnt_ref[...] = zeros
        acc_ref[...] = zeros
        cfs_ref[...] = zeros

    best = sm_ref[0]
    sbest = jnp.full(best.shape, float(c_dim - 1), jnp.float32)
    for c in range(1, c_dim):
        xc = sm_ref[c]
        gt = xc > best
        best = jnp.maximum(xc, best)
        sbest = jnp.where(gt, float(c_dim - 1 - c), sbest)

    target = float(c_dim - 1) - lab_ref[...].astype(jnp.float32)
    corr = (sbest == target).astype(jnp.float32)

    for i in range(_NUM_BINS + 1):
        m = best > bnd_ref[i]
        cnt_ref[i] += _lane_fold(m.astype(jnp.float32))
        acc_ref[i] += _lane_fold(jnp.where(m, corr, 0.0))
        cfs_ref[i] += _lane_fold(jnp.where(m, best, 0.0))

    @pl.when(j == nb - 1)
    def _fin():
        cnt = jnp.sum(cnt_ref[...], axis=2)
        accs = jnp.sum(acc_ref[...], axis=2)
        cfss = jnp.sum(cfs_ref[...], axis=2)
        count = cnt[:-1] - cnt[1:]
        prop = count / float(n_total)
        denom = jnp.maximum(count, 1.0)
        acc_b = (accs[:-1] - accs[1:]) / denom
        cfs_b = (cfss[:-1] - cfss[1:]) / denom
        contrib = jnp.where(count > 0.0,
                            jnp.abs(cfs_b - acc_b) * prop, 0.0)
        ece = jnp.sum(contrib, axis=0)
        out_ref[...] = jnp.broadcast_to(ece[:, None], out_ref.shape)


def kernel(edl_u, softmax, label):
    del edl_u
    b_dim, c_dim, n = softmax.shape
    sm_t = jnp.transpose(softmax, (1, 0, 2))
    tile = 8192
    while n % tile:
        tile //= 2
    nb = n // tile

    label = label.astype(jnp.int32)
    bnd = jnp.linspace(0.0, 1.0, _NUM_BINS + 1, dtype=jnp.float32)

    body = functools.partial(_ece_body, nb, c_dim, n)
    out = pl.pallas_call(
        body,
        grid=(nb,),
        in_specs=[
            pl.BlockSpec(memory_space=pltpu.SMEM),
            pl.BlockSpec((c_dim, b_dim, tile), lambda j: (0, 0, j)),
            pl.BlockSpec((b_dim, tile), lambda j: (0, j)),
        ],
        out_specs=pl.BlockSpec((b_dim, _LANES), lambda j: (0, 0)),
        out_shape=jax.ShapeDtypeStruct((b_dim, _LANES), jnp.float32),
        scratch_shapes=[pltpu.VMEM((_NUM_BINS + 1, b_dim, _LANES),
                                   jnp.float32) for _ in range(3)],
    )(bnd, sm_t, label)
    return out[:, 0]
